# Initial kernel scaffold; baseline (speedup 1.0000x reference)
import jax
import jax.numpy as jnp
from jax.experimental import pallas as pl


def kernel(top_k):
    raise NotImplementedError("write your pallas kernel here")


# speedup vs baseline: 86.5465x; 86.5465x over previous
import functools

import jax
import jax.numpy as jnp
from jax.experimental import pallas as pl
from jax.experimental.pallas import tpu as pltpu

_K = 8

_SORT8 = (
    (0, 1), (2, 3), (4, 5), (6, 7),
    (0, 2), (1, 3), (4, 6), (5, 7),
    (1, 2), (5, 6),
    (0, 4), (1, 5), (2, 6), (3, 7),
    (2, 4), (3, 5),
    (1, 2), (3, 4), (5, 6),
)

_BITONIC8 = (
    (0, 4), (1, 5), (2, 6), (3, 7),
    (0, 2), (1, 3), (4, 6), (5, 7),
    (0, 1), (2, 3), (4, 5), (6, 7),
)


def _cas(s, net):
    for i, j in net:
        a, b = s[i], s[j]
        s[i] = jnp.maximum(a, b)
        s[j] = jnp.minimum(a, b)
    return s


def _merge8(a, b):
    top = [jnp.maximum(a[i], b[_K - 1 - i]) for i in range(_K)]
    return _cas(top, _BITONIC8)


def _chunk_top8(x, chunk):
    r = chunk // _K
    s = [x[i * r:(i + 1) * r, :] for i in range(_K)]
    s = _cas(s, _SORT8)
    while r > 1:
        h = r // 2
        s = _merge8([v[:h, :] for v in s], [v[h:, :] for v in s])
        r = h
    return s


def _kmax_body(x_ref, o_ref, acc_ref, *, chunk, c):
    t = pl.program_id(1)
    nt = pl.num_programs(1)

    s = _chunk_top8(x_ref[0], chunk)

    @pl.when(t == 0)
    def _init():
        acc_ref[...] = jnp.full((_K, c), -jnp.inf, dtype=jnp.float32)

    acc = acc_ref[...]
    merged = _merge8([acc[i:i + 1, :] for i in range(_K)], s)
    out = jnp.concatenate(merged, axis=0)
    acc_ref[...] = out

    @pl.when(t == nt - 1)
    def _emit():
        o_ref[0] = out


def kernel(top_k):
    b, t, c = top_k.shape
    chunk = 1024
    while t % chunk != 0:
        chunk //= 2
    nt = t // chunk

    body = functools.partial(_kmax_body, chunk=chunk, c=c)
    return pl.pallas_call(
        body,
        grid=(b, nt),
        in_specs=[pl.BlockSpec((1, chunk, c), lambda bi, ti: (bi, ti, 0))],
        out_specs=pl.BlockSpec((1, _K, c), lambda bi, ti: (bi, 0, 0)),
        out_shape=jax.ShapeDtypeStruct((b, _K, c), jnp.float32),
        scratch_shapes=[pltpu.VMEM((_K, c), jnp.float32)],
        compiler_params=pltpu.CompilerParams(
            dimension_semantics=("parallel", "arbitrary")),
    )(top_k)


# speedup vs baseline: 145.9925x; 1.6869x over previous
import functools

import jax
import jax.numpy as jnp
from jax.experimental import pallas as pl
from jax.experimental.pallas import tpu as pltpu

_K = 8
_GROUP = 64

_SORT8 = (
    (0, 1), (2, 3), (4, 5), (6, 7),
    (0, 2), (1, 3), (4, 6), (5, 7),
    (1, 2), (5, 6),
    (0, 4), (1, 5), (2, 6), (3, 7),
    (2, 4), (3, 5),
    (1, 2), (3, 4), (5, 6),
)

_BITONIC8 = (
    (0, 4), (1, 5), (2, 6), (3, 7),
    (0, 2), (1, 3), (4, 6), (5, 7),
    (0, 1), (2, 3), (4, 5), (6, 7),
)


def _cas(s, net):
    s = list(s)
    for i, j in net:
        a, b = s[i], s[j]
        s[i] = jnp.maximum(a, b)
        s[j] = jnp.minimum(a, b)
    return s


def _merge8(a, b):
    top = [jnp.maximum(a[i], b[_K - 1 - i]) for i in range(_K)]
    return _cas(top, _BITONIC8)


def _kmax_body(x_ref, o_ref, acc_ref, *, chunk, c, cw):
    t = pl.program_id(1)
    nt = pl.num_programs(1)
    ngroups = chunk // _GROUP

    @pl.when(t == 0)
    def _init():
        acc_ref[...] = jnp.full((_K, _K, c), -jnp.inf, dtype=jnp.float32)

    for half in range(c // cw):
        lanes = slice(half * cw, (half + 1) * cw)

        def _group(g, acc, lanes=lanes):
            x = x_ref[0, pl.ds(g * _GROUP, _GROUP), lanes]
            s = [x[k * _K:(k + 1) * _K, :] for k in range(_K)]
            return tuple(_merge8(list(acc), _cas(s, _SORT8)))

        acc = tuple(acc_ref[k, :, lanes] for k in range(_K))
        acc = jax.lax.fori_loop(0, ngroups, _group, acc, unroll=2)
        for k in range(_K):
            acc_ref[k, :, lanes] = acc[k]

    @pl.when(t == nt - 1)
    def _emit():
        a = [acc_ref[k, :, :] for k in range(_K)]
        for shift in (4, 2, 1):
            rolled = [pltpu.roll(v, shift, 0) for v in a]
            a = _merge8(a, rolled)
        o_ref[0] = jnp.concatenate([v[0:1, :] for v in a], axis=0)


def kernel(top_k):
    b, t, c = top_k.shape
    chunk = 1024
    while t % chunk != 0:
        chunk //= 2
    nt = t // chunk
    cw = c // 2 if c % 256 == 0 else c

    body = functools.partial(_kmax_body, chunk=chunk, c=c, cw=cw)
    return pl.pallas_call(
        body,
        grid=(b, nt),
        in_specs=[pl.BlockSpec((1, chunk, c), lambda bi, ti: (bi, ti, 0))],
        out_specs=pl.BlockSpec((1, _K, c), lambda bi, ti: (bi, 0, 0)),
        out_shape=jax.ShapeDtypeStruct((b, _K, c), jnp.float32),
        scratch_shapes=[pltpu.VMEM((_K, _K, c), jnp.float32)],
        compiler_params=pltpu.CompilerParams(
            dimension_semantics=("parallel", "arbitrary")),
    )(top_k)


# speedup vs baseline: 163.3487x; 1.1189x over previous
import functools

import jax
import jax.numpy as jnp
from jax.experimental import pallas as pl
from jax.experimental.pallas import tpu as pltpu

_K = 8
_GROUP = 64

_SORT8 = (
    (0, 1), (2, 3), (4, 5), (6, 7),
    (0, 2), (1, 3), (4, 6), (5, 7),
    (1, 2), (5, 6),
    (0, 4), (1, 5), (2, 6), (3, 7),
    (2, 4), (3, 5),
    (1, 2), (3, 4), (5, 6),
)

_BITONIC8 = (
    (0, 4), (1, 5), (2, 6), (3, 7),
    (0, 2), (1, 3), (4, 6), (5, 7),
    (0, 1), (2, 3), (4, 5), (6, 7),
)


def _cas(s, net):
    s = list(s)
    for i, j in net:
        a, b = s[i], s[j]
        s[i] = jnp.maximum(a, b)
        s[j] = jnp.minimum(a, b)
    return s


def _merge8(a, b):
    top = [jnp.maximum(a[i], b[_K - 1 - i]) for i in range(_K)]
    return _cas(top, _BITONIC8)


def _kmax_body(x_ref, o_ref, acc_ref, *, chunk, c, cw):
    t = pl.program_id(1)
    nt = pl.num_programs(1)
    ngroups = chunk // _GROUP

    @pl.when(t == 0)
    def _init():
        acc_ref[...] = jnp.full((_K, _K, c), -jnp.inf, dtype=jnp.float32)

    for half in range(c // cw):
        lanes = slice(half * cw, (half + 1) * cw)

        def _group(g, acc, lanes=lanes):
            x = x_ref[0, pl.ds(g * _GROUP, _GROUP), lanes]
            s = [x[k * _K:(k + 1) * _K, :] for k in range(_K)]
            return tuple(_merge8(list(acc), _cas(s, _SORT8)))

        acc = tuple(acc_ref[k, :, lanes] for k in range(_K))
        acc = jax.lax.fori_loop(0, ngroups, _group, acc, unroll=2)
        for k in range(_K):
            acc_ref[k, :, lanes] = acc[k]

    @pl.when(t == nt - 1)
    def _emit():
        a = [acc_ref[k, :, :] for k in range(_K)]
        for shift in (4, 2, 1):
            rolled = [pltpu.roll(v, shift, 0) for v in a]
            a = _merge8(a, rolled)
        o_ref[0] = jnp.concatenate([v[0:1, :] for v in a], axis=0)


def kernel(top_k):
    b, t, c = top_k.shape
    chunk = 2048
    while t % chunk != 0:
        chunk //= 2
    nt = t // chunk
    cw = c // 2 if c % 256 == 0 else c

    body = functools.partial(_kmax_body, chunk=chunk, c=c, cw=cw)
    return pl.pallas_call(
        body,
        grid=(b, nt),
        in_specs=[pl.BlockSpec((1, chunk, c), lambda bi, ti: (bi, ti, 0))],
        out_specs=pl.BlockSpec((1, _K, c), lambda bi, ti: (bi, 0, 0)),
        out_shape=jax.ShapeDtypeStruct((b, _K, c), jnp.float32),
        scratch_shapes=[pltpu.VMEM((_K, _K, c), jnp.float32)],
        compiler_params=pltpu.CompilerParams(
            dimension_semantics=("parallel", "arbitrary")),
    )(top_k)


# speedup vs baseline: 175.7923x; 1.0762x over previous
import functools

import jax
import jax.numpy as jnp
from jax.experimental import pallas as pl
from jax.experimental.pallas import tpu as pltpu

_K = 8
_GROUP = 64

_SORT8 = (
    (0, 1), (2, 3), (4, 5), (6, 7),
    (0, 2), (1, 3), (4, 6), (5, 7),
    (1, 2), (5, 6),
    (0, 4), (1, 5), (2, 6), (3, 7),
    (2, 4), (3, 5),
    (1, 2), (3, 4), (5, 6),
)

_BITONIC8 = (
    (0, 4), (1, 5), (2, 6), (3, 7),
    (0, 2), (1, 3), (4, 6), (5, 7),
    (0, 1), (2, 3), (4, 5), (6, 7),
)


def _cas(s, net):
    s = list(s)
    for i, j in net:
        a, b = s[i], s[j]
        s[i] = jnp.maximum(a, b)
        s[j] = jnp.minimum(a, b)
    return s


def _merge8(a, b):
    top = [jnp.maximum(a[i], b[_K - 1 - i]) for i in range(_K)]
    return _cas(top, _BITONIC8)


def _kmax_body(x_ref, o_ref, acc_ref, *, chunk, c, cw):
    t = pl.program_id(1)
    nt = pl.num_programs(1)
    ngroups = chunk // _GROUP

    @pl.when(t == 0)
    def _init():
        acc_ref[...] = jnp.full((_K, _K, c), -jnp.inf, dtype=jnp.float32)

    for half in range(c // cw):
        lanes = slice(half * cw, (half + 1) * cw)

        def _group(g, acc, lanes=lanes):
            x = x_ref[0, pl.ds(g * _GROUP, _GROUP), lanes]
            s = [x[k * _K:(k + 1) * _K, :] for k in range(_K)]
            return tuple(_merge8(list(acc), _cas(s, _SORT8)))

        acc = tuple(acc_ref[k, :, lanes] for k in range(_K))
        acc = jax.lax.fori_loop(0, ngroups, _group, acc, unroll=4)
        for k in range(_K):
            acc_ref[k, :, lanes] = acc[k]

    @pl.when(t == nt - 1)
    def _emit():
        a = [acc_ref[k, :, :] for k in range(_K)]
        for shift in (4, 2, 1):
            rolled = [pltpu.roll(v, shift, 0) for v in a]
            a = _merge8(a, rolled)
        o_ref[0] = jnp.concatenate([v[0:1, :] for v in a], axis=0)


def kernel(top_k):
    b, t, c = top_k.shape
    chunk = 4096
    while t % chunk != 0:
        chunk //= 2
    nt = t // chunk
    cw = c // 2 if c % 256 == 0 else c

    body = functools.partial(_kmax_body, chunk=chunk, c=c, cw=cw)
    return pl.pallas_call(
        body,
        grid=(b, nt),
        in_specs=[pl.BlockSpec((1, chunk, c), lambda bi, ti: (bi, ti, 0))],
        out_specs=pl.BlockSpec((1, _K, c), lambda bi, ti: (bi, 0, 0)),
        out_shape=jax.ShapeDtypeStruct((b, _K, c), jnp.float32),
        scratch_shapes=[pltpu.VMEM((_K, _K, c), jnp.float32)],
        compiler_params=pltpu.CompilerParams(
            dimension_semantics=("parallel", "arbitrary")),
    )(top_k)
